# baseline (speedup 1.0000x reference)
import jax
import jax.numpy as jnp
from jax import lax
from jax.experimental import pallas as pl
from jax.experimental.pallas import tpu as pltpu
from jax.experimental.pallas import tpu_sc as plsc

_B = 16384
_D = 32
_W = 128
_RPB = _W // _D
_NC = 2
_NS = 16
_NW = _NC * _NS
_BPW = _B // _NW
_CH = 128
_NCH = _BPW // _CH
_L = 16


def _body(z_hbm, rows_hbm, cols_hbm, out_hbm, ridx, cidx, rblk, cblk,
          zr, zc, outv, sem):
    wid = lax.axis_index("s") * _NC + lax.axis_index("c")
    base = wid * _BPW

    for j in range(_NCH):
        pltpu.sync_copy(rows_hbm.at[pl.ds(base + j * _CH, _CH)], ridx.at[j])
        pltpu.sync_copy(cols_hbm.at[pl.ds(base + j * _CH, _CH)], cidx.at[j])

    for j in range(_NCH):
        for k in range(_CH // _L):
            s = pl.ds(k * _L, _L)
            rblk[j, s] = lax.shift_right_logical(ridx[j, s], 2)
            cblk[j, s] = lax.shift_right_logical(cidx[j, s], 2)

    def chunk(j, zr_j, zc_j):
        cp1 = pltpu.async_copy(z_hbm.at[rblk.at[j]], zr_j, sem)
        cp2 = pltpu.async_copy(z_hbm.at[cblk.at[j]], zc_j, sem)
        cp1.wait()
        cp2.wait()

        def group(k, carry):
            s = pl.ds(k * _L, _L)
            rowi = lax.iota(jnp.int32, _L) + k * _L
            ro = lax.shift_left(lax.bitwise_and(ridx[j, s], _RPB - 1), 5)
            co = lax.shift_left(lax.bitwise_and(cidx[j, s], _RPB - 1), 5)
            acc = jnp.zeros((_L,), jnp.float32)
            for d in range(_D):
                r = plsc.load_gather(zr_j, [rowi, ro + d])
                c = plsc.load_gather(zc_j, [rowi, co + d])
                acc = acc + r * c
            outv[pl.ds(j * _CH + k * _L, _L)] = acc
            return carry

        lax.fori_loop(0, _CH // _L, group, 0)

    for j in range(_NCH):
        chunk(j, zr, zc)

    pltpu.sync_copy(outv, out_hbm.at[pl.ds(base, _BPW)])


def kernel(z, drp_rows, drp_cols):
    zb = z.reshape(z.shape[0] * _D // _W, _W)
    mesh = plsc.VectorSubcoreMesh(core_axis_name="c", subcore_axis_name="s")
    f = pl.kernel(
        _body,
        out_type=jax.ShapeDtypeStruct((_B,), jnp.float32),
        mesh=mesh,
        compiler_params=pltpu.CompilerParams(
            needs_layout_passes=False, use_tc_tiling_on_sc=True),
        scratch_types=[
            pltpu.VMEM((_NCH, _CH), jnp.int32),
            pltpu.VMEM((_NCH, _CH), jnp.int32),
            pltpu.VMEM((_NCH, _CH), jnp.int32),
            pltpu.VMEM((_NCH, _CH), jnp.int32),
            pltpu.VMEM((_CH, _W), jnp.float32),
            pltpu.VMEM((_CH, _W), jnp.float32),
            pltpu.VMEM((_BPW,), jnp.float32),
            pltpu.SemaphoreType.DMA,
        ],
    )
    return f(zb, drp_rows.astype(jnp.int32), drp_cols.astype(jnp.int32))
